# baseline (speedup 1.0000x reference)
import functools

import jax
import jax.numpy as jnp
from jax import lax
from jax.experimental import pallas as pl
from jax.experimental.pallas import tpu as pltpu
from jax.experimental.pallas import tpu_sc as plsc

N_ROWS = 1000000
D = 32
L2_REG = 1e-4
B = 16384

NC = 2
NS = 16
L = 16
NW = NC * NS
BPW = B // NW
GPW = BPW // L
E = BPW * D
CHUNK = 128


def _sc_body(uidx, pidx, nidx, uflat, iflat, diff_out, reg_out,
             stg_u, stg_p, stg_n, ix_u, ix_p, ix_n, g_u, g_p, g_n,
             diff_s, regbuf, sem):
    wid = lax.axis_index("s") * NC + lax.axis_index("c")
    base = wid * BPW

    pltpu.sync_copy(uidx.at[pl.ds(base, BPW)], stg_u)
    pltpu.sync_copy(pidx.at[pl.ds(base, BPW)], stg_p)
    pltpu.sync_copy(nidx.at[pl.ds(base, BPW)], stg_n)

    def build(g, c):
        u16 = stg_u[pl.ds(g * L, L)]
        p16 = stg_p[pl.ds(g * L, L)]
        n16 = stg_n[pl.ds(g * L, L)]
        for d in range(D):
            off = d * BPW + g * L
            ix_u[pl.ds(off, L)] = u16 + (d * N_ROWS)
            ix_p[pl.ds(off, L)] = p16 + (d * N_ROWS)
            ix_n[pl.ds(off, L)] = n16 + (d * N_ROWS)
        return c

    lax.fori_loop(0, GPW, build, 0)

    for c in range(E // CHUNK):
        sl = pl.ds(c * CHUNK, CHUNK)
        pltpu.async_copy(uflat.at[ix_u.at[sl]], g_u.at[sl], sem)
        pltpu.async_copy(iflat.at[ix_p.at[sl]], g_p.at[sl], sem)
        pltpu.async_copy(iflat.at[ix_n.at[sl]], g_n.at[sl], sem)
    pltpu.make_async_copy(uflat.at[pl.ds(0, E)], g_u, sem).wait()
    pltpu.make_async_copy(iflat.at[pl.ds(0, E)], g_p, sem).wait()
    pltpu.make_async_copy(iflat.at[pl.ds(0, E)], g_n, sem).wait()

    def group(g, regacc):
        acc = jnp.zeros((L,), jnp.float32)
        for d in range(D):
            off = d * BPW + g * L
            au = g_u[pl.ds(off, L)]
            ap = g_p[pl.ds(off, L)]
            an = g_n[pl.ds(off, L)]
            acc = acc + au * (ap - an)
            regacc = regacc + (au * au + ap * ap + an * an)
        diff_s[pl.ds(g * L, L)] = acc
        return regacc

    regv = lax.fori_loop(0, GPW, group, jnp.zeros((L,), jnp.float32))

    pltpu.sync_copy(diff_s, diff_out.at[pl.ds(base, BPW)])
    regbuf[pl.ds(0, L)] = regv
    pltpu.sync_copy(regbuf, reg_out.at[wid])


@jax.jit
def _sc_call(uidx, pidx, nidx, uflat, iflat):
    mesh = plsc.VectorSubcoreMesh(core_axis_name="c", subcore_axis_name="s",
                                  num_cores=NC, num_subcores=NS)
    f = functools.partial(
        pl.kernel,
        out_type=(jax.ShapeDtypeStruct((B,), jnp.float32),
                  jax.ShapeDtypeStruct((NW, L), jnp.float32)),
        mesh=mesh,
        compiler_params=pltpu.CompilerParams(use_tc_tiling_on_sc=False),
        scratch_types=[
            pltpu.VMEM((BPW,), jnp.int32),
            pltpu.VMEM((BPW,), jnp.int32),
            pltpu.VMEM((BPW,), jnp.int32),
            pltpu.VMEM((E,), jnp.int32),
            pltpu.VMEM((E,), jnp.int32),
            pltpu.VMEM((E,), jnp.int32),
            pltpu.VMEM((E,), jnp.float32),
            pltpu.VMEM((E,), jnp.float32),
            pltpu.VMEM((E,), jnp.float32),
            pltpu.VMEM((BPW,), jnp.float32),
            pltpu.VMEM((L,), jnp.float32),
            pltpu.SemaphoreType.DMA,
        ],
    )(_sc_body)
    return f(uidx, pidx, nidx, uflat, iflat)


def _tc_body(diff_ref, reg_ref, out_ref):
    x = diff_ref[...]
    loss = -jnp.mean(jnp.log(jax.nn.sigmoid(x) + 1e-8))
    reg = jnp.sum(reg_ref[...])
    out_ref[0, 0] = loss + L2_REG * (reg / B)


@jax.jit
def _tc_call(diff, regpart):
    out = pl.pallas_call(
        _tc_body,
        out_shape=jax.ShapeDtypeStruct((1, 1), jnp.float32),
        out_specs=pl.BlockSpec(memory_space=pltpu.SMEM),
    )(diff.reshape(128, 128), regpart.reshape(4, 128))
    return out.reshape(())


def kernel(user_indices, pos_item_indices, neg_item_indices,
           user_embedding, item_embedding):
    uflat = user_embedding.T.reshape(-1)
    iflat = item_embedding.T.reshape(-1)
    diff, regpart = _sc_call(user_indices, pos_item_indices, neg_item_indices,
                             uflat, iflat)
    return _tc_call(diff, regpart)


# speedup vs baseline: 2.7210x; 2.7210x over previous
import functools

import jax
import jax.numpy as jnp
from jax import lax
from jax.experimental import pallas as pl
from jax.experimental.pallas import tpu as pltpu
from jax.experimental.pallas import tpu_sc as plsc

D = 32
L2_REG = 1e-4
B = 16384

NC = 2
NS = 16
L = 16
NW = NC * NS
BPW = B // NW
CHUNK = 128
NCHUNK = BPW // CHUNK


def _sc_body(uidx, pidx, nidx, uemb, iemb, pvec_out, reg_out,
             idx_u, idx_p, idx_n, rows_u, rows_p, rows_n, pout, regbuf, sem):
    wid = lax.axis_index("s") * NC + lax.axis_index("c")
    base = wid * BPW

    pltpu.sync_copy(uidx.at[pl.ds(base, BPW)], idx_u)
    pltpu.sync_copy(pidx.at[pl.ds(base, BPW)], idx_p)
    pltpu.sync_copy(nidx.at[pl.ds(base, BPW)], idx_n)

    descs = []
    for j in range(NCHUNK):
        sl = pl.ds(j * CHUNK, CHUNK)
        descs.append(pltpu.async_copy(uemb.at[idx_u.at[sl]], rows_u.at[sl], sem))
        descs.append(pltpu.async_copy(iemb.at[idx_p.at[sl]], rows_p.at[sl], sem))
        descs.append(pltpu.async_copy(iemb.at[idx_n.at[sl]], rows_n.at[sl], sem))
    for d in descs:
        d.wait()

    def row_step(r, regacc):
        u0 = rows_u[r, pl.ds(0, L)]
        u1 = rows_u[r, pl.ds(L, L)]
        p0 = rows_p[r, pl.ds(0, L)]
        p1 = rows_p[r, pl.ds(L, L)]
        n0 = rows_n[r, pl.ds(0, L)]
        n1 = rows_n[r, pl.ds(L, L)]
        pout[r, pl.ds(0, L)] = u0 * (p0 - n0) + u1 * (p1 - n1)
        return regacc + (u0 * u0 + u1 * u1 + p0 * p0 + p1 * p1
                         + n0 * n0 + n1 * n1)

    regv = lax.fori_loop(0, BPW, row_step, jnp.zeros((L,), jnp.float32))

    pltpu.sync_copy(pout, pvec_out.at[pl.ds(base, BPW)])
    regbuf[pl.ds(0, L)] = regv
    pltpu.sync_copy(regbuf, reg_out.at[wid])


@jax.jit
def _sc_call(uidx, pidx, nidx, uemb, iemb):
    mesh = plsc.VectorSubcoreMesh(core_axis_name="c", subcore_axis_name="s",
                                  num_cores=NC, num_subcores=NS)
    f = functools.partial(
        pl.kernel,
        out_type=(jax.ShapeDtypeStruct((B, L), jnp.float32),
                  jax.ShapeDtypeStruct((NW, L), jnp.float32)),
        mesh=mesh,
        compiler_params=pltpu.CompilerParams(use_tc_tiling_on_sc=False),
        scratch_types=[
            pltpu.VMEM((BPW,), jnp.int32),
            pltpu.VMEM((BPW,), jnp.int32),
            pltpu.VMEM((BPW,), jnp.int32),
            pltpu.VMEM((BPW, D), jnp.float32),
            pltpu.VMEM((BPW, D), jnp.float32),
            pltpu.VMEM((BPW, D), jnp.float32),
            pltpu.VMEM((BPW, L), jnp.float32),
            pltpu.VMEM((L,), jnp.float32),
            pltpu.SemaphoreType.DMA,
        ],
    )(_sc_body)
    return f(uidx, pidx, nidx, uemb, iemb)


def _tr_body(src_ref, out_ref):
    out_ref[...] = src_ref[...].T


@jax.jit
def _tc_transpose(tabT):
    n = tabT.shape[1]
    blk = 512
    grid = (n + blk - 1) // blk
    return pl.pallas_call(
        _tr_body,
        grid=(grid,),
        in_specs=[pl.BlockSpec((D, blk), lambda i: (0, i))],
        out_specs=pl.BlockSpec((blk, D), lambda i: (i, 0)),
        out_shape=jax.ShapeDtypeStruct((n, D), jnp.float32),
    )(tabT)


def _tc_body(pvec_ref, reg_ref, out_ref):
    x = pvec_ref[...]
    c = lax.broadcasted_iota(jnp.int32, (128, 8), 0)
    j = lax.broadcasted_iota(jnp.int32, (128, 8), 1)
    sel = jnp.where(c // L == j, 1.0, 0.0).astype(jnp.float32)
    diff = jax.lax.dot_general(x, sel, (((1,), (0,)), ((), ())),
                               preferred_element_type=jnp.float32)
    loss = -jnp.mean(jnp.log(jax.nn.sigmoid(diff) + 1e-8))
    reg = jnp.sum(reg_ref[...])
    out_ref[0, 0] = loss + L2_REG * (reg / B)


@jax.jit
def _tc_call(pvec, regpart):
    out = pl.pallas_call(
        _tc_body,
        out_shape=jax.ShapeDtypeStruct((1, 1), jnp.float32),
        out_specs=pl.BlockSpec(memory_space=pltpu.SMEM),
    )(pvec.reshape(2048, 128), regpart.reshape(4, 128))
    return out.reshape(())


def kernel(user_indices, pos_item_indices, neg_item_indices,
           user_embedding, item_embedding):
    iemb_rm = _tc_transpose(item_embedding.T)
    pvec, regpart = _sc_call(user_indices, pos_item_indices, neg_item_indices,
                             user_embedding, iemb_rm)
    return _tc_call(pvec, regpart)


# speedup vs baseline: 5.7329x; 2.1069x over previous
import functools

import jax
import jax.numpy as jnp
from jax import lax
from jax.experimental import pallas as pl
from jax.experimental.pallas import tpu as pltpu
from jax.experimental.pallas import tpu_sc as plsc

D = 32
L2_REG = 1e-4
B = 16384

NC = 2
NS = 16
L = 16
NW = NC * NS
BPW = B // NW
CHUNK = 128
NCHUNK = BPW // CHUNK


def _sc_body(uidx, pidx, nidx, uemb, iemb, pvec_out, reg_out,
             idx_u, idx_p, idx_n, rows_u, rows_p, rows_n, pout, regbuf, sem):
    wid = lax.axis_index("s") * NC + lax.axis_index("c")
    base = wid * BPW

    pltpu.sync_copy(uidx.at[pl.ds(base, BPW)], idx_u)
    pltpu.sync_copy(pidx.at[pl.ds(base, BPW)], idx_p)
    pltpu.sync_copy(nidx.at[pl.ds(base, BPW)], idx_n)

    descs = []
    for j in range(NCHUNK):
        sl = pl.ds(j * CHUNK, CHUNK)
        descs.append(pltpu.async_copy(uemb.at[idx_u.at[sl]], rows_u.at[sl], sem))
        descs.append(pltpu.async_copy(iemb.at[idx_p.at[sl]], rows_p.at[sl], sem))
        descs.append(pltpu.async_copy(iemb.at[idx_n.at[sl]], rows_n.at[sl], sem))
    for d in descs:
        d.wait()

    def row_step(r, regacc):
        u0 = rows_u[r, pl.ds(0, L)]
        u1 = rows_u[r, pl.ds(L, L)]
        p0 = rows_p[r, pl.ds(0, L)]
        p1 = rows_p[r, pl.ds(L, L)]
        n0 = rows_n[r, pl.ds(0, L)]
        n1 = rows_n[r, pl.ds(L, L)]
        pout[r, pl.ds(0, L)] = u0 * (p0 - n0) + u1 * (p1 - n1)
        return regacc + (u0 * u0 + u1 * u1 + p0 * p0 + p1 * p1
                         + n0 * n0 + n1 * n1)

    regv = lax.fori_loop(0, BPW, row_step, jnp.zeros((L,), jnp.float32))

    pltpu.sync_copy(pout, pvec_out.at[pl.ds(base, BPW)])
    regbuf[pl.ds(0, L)] = regv
    pltpu.sync_copy(regbuf, reg_out.at[wid])


@jax.jit
def _sc_call(uidx, pidx, nidx, uemb, iemb):
    mesh = plsc.VectorSubcoreMesh(core_axis_name="c", subcore_axis_name="s",
                                  num_cores=NC, num_subcores=NS)
    f = functools.partial(
        pl.kernel,
        out_type=(jax.ShapeDtypeStruct((B, L), jnp.float32),
                  jax.ShapeDtypeStruct((NW, L), jnp.float32)),
        mesh=mesh,
        compiler_params=pltpu.CompilerParams(use_tc_tiling_on_sc=False),
        scratch_types=[
            pltpu.VMEM((BPW,), jnp.int32),
            pltpu.VMEM((BPW,), jnp.int32),
            pltpu.VMEM((BPW,), jnp.int32),
            pltpu.VMEM((BPW, D), jnp.float32),
            pltpu.VMEM((BPW, D), jnp.float32),
            pltpu.VMEM((BPW, D), jnp.float32),
            pltpu.VMEM((BPW, L), jnp.float32),
            pltpu.VMEM((L,), jnp.float32),
            pltpu.SemaphoreType.DMA,
        ],
    )(_sc_body)
    return f(uidx, pidx, nidx, uemb, iemb)


def _tc_body(pvec_ref, reg_ref, out_ref):
    x = pvec_ref[...]
    c = lax.broadcasted_iota(jnp.int32, (128, 8), 0)
    j = lax.broadcasted_iota(jnp.int32, (128, 8), 1)
    sel = jnp.where(c // L == j, 1.0, 0.0).astype(jnp.float32)
    diff = jax.lax.dot_general(x, sel, (((1,), (0,)), ((), ())),
                               preferred_element_type=jnp.float32)
    loss = -jnp.mean(jnp.log(jax.nn.sigmoid(diff) + 1e-8))
    reg = jnp.sum(reg_ref[...])
    out_ref[0, 0] = loss + L2_REG * (reg / B)


@jax.jit
def _tc_call(pvec, regpart):
    out = pl.pallas_call(
        _tc_body,
        out_shape=jax.ShapeDtypeStruct((1, 1), jnp.float32),
        out_specs=pl.BlockSpec(memory_space=pltpu.SMEM),
    )(pvec.reshape(2048, 128), regpart.reshape(4, 128))
    return out.reshape(())


def kernel(user_indices, pos_item_indices, neg_item_indices,
           user_embedding, item_embedding):
    pvec, regpart = _sc_call(user_indices, pos_item_indices, neg_item_indices,
                             user_embedding, item_embedding)
    return _tc_call(pvec, regpart)
